# Initial kernel scaffold; baseline (speedup 1.0000x reference)
import jax
import jax.numpy as jnp
from jax.experimental import pallas as pl


def kernel(x, edge_index, batch, W1, b1, W2, b2, W3, b3):
    raise NotImplementedError("write your pallas kernel here")


# speedup vs baseline: 22.2533x; 22.2533x over previous
import functools

import jax
import jax.numpy as jnp
from jax import lax
from jax.experimental import pallas as pl
from jax.experimental.pallas import tpu as pltpu
from jax.experimental.pallas import tpu_sc as plsc

N = 100000
E = 1600000
NSEG = 256
F1 = 16
F2 = 32

NTEC = 16
NPAD = 100096
SLICE = NPAD // NTEC
ROW = 80
NROWS = E // ROW

BLK = 2000
GRID = N // BLK


def _mesh():
    return plsc.VectorSubcoreMesh(core_axis_name="c", subcore_axis_name="s")


_SC_PARAMS = pltpu.CompilerParams(use_tc_tiling_on_sc=False)


def _sc_count(dst2d, ones_row, zslice1):
    @functools.partial(
        pl.kernel,
        out_type=jax.ShapeDtypeStruct((2, NPAD, F1), jnp.float32),
        mesh=_mesh(),
        compiler_params=_SC_PARAMS,
        scratch_types=[
            pltpu.VMEM((5, ROW), jnp.int32),
            pltpu.VMEM((ROW, F1), jnp.float32),
            pltpu.VMEM_SHARED((NPAD, F1), jnp.float32),
        ],
    )
    def k(dst_hbm, ones_hbm, z_hbm, out_hbm, di_v, ones_v, cnt_sh):
        c = lax.axis_index("c")
        s = lax.axis_index("s")
        pltpu.sync_copy(ones_hbm, ones_v)
        pltpu.sync_copy(z_hbm, cnt_sh.at[pl.ds(s * SLICE, SLICE), :])
        plsc.subcore_barrier()
        base = (c * NTEC + s) * (NROWS // 32)

        def outer(t, carry):
            pltpu.sync_copy(dst_hbm.at[pl.ds(base + t * 5, 5), :], di_v)
            for j in range(5):
                pltpu.sync_copy(ones_v, cnt_sh.at[di_v.at[j]], add=True)
            return carry

        lax.fori_loop(0, (NROWS // 32) // 5, outer, 0)
        plsc.subcore_barrier()
        pltpu.sync_copy(cnt_sh.at[pl.ds(s * SLICE, SLICE), :],
                        out_hbm.at[c, pl.ds(s * SLICE, SLICE), :])

    return k(dst2d, ones_row, zslice1)


def _sc_aggregate1(g1, src2d, dst2d, zslice16):
    @functools.partial(
        pl.kernel,
        out_type=jax.ShapeDtypeStruct((2, NPAD, F1), jnp.float32),
        mesh=_mesh(),
        compiler_params=_SC_PARAMS,
        scratch_types=[
            pltpu.VMEM((5, ROW), jnp.int32),
            pltpu.VMEM((5, ROW), jnp.int32),
            pltpu.VMEM((5, ROW, F1), jnp.float32),
            pltpu.VMEM_SHARED((NPAD, F1), jnp.float32),
            pltpu.SemaphoreType.DMA,
        ],
    )
    def k(g_hbm, src_hbm, dst_hbm, z_hbm, out_hbm,
          si_v, di_v, rows_v, acc_sh, sem):
        c = lax.axis_index("c")
        s = lax.axis_index("s")
        pltpu.sync_copy(z_hbm, acc_sh.at[pl.ds(s * SLICE, SLICE), :])
        plsc.subcore_barrier()
        base = (c * NTEC + s) * (NROWS // 32)

        def outer(t, carry):
            r0 = base + t * 5
            pltpu.sync_copy(src_hbm.at[pl.ds(r0, 5), :], si_v)
            pltpu.sync_copy(dst_hbm.at[pl.ds(r0, 5), :], di_v)
            cps = [pltpu.async_copy(g_hbm.at[si_v.at[j]], rows_v.at[j], sem)
                   for j in range(5)]
            for j in range(5):
                cps[j].wait()
                pltpu.sync_copy(rows_v.at[j], acc_sh.at[di_v.at[j]], add=True)
            return carry

        lax.fori_loop(0, (NROWS // 32) // 5, outer, 0)
        plsc.subcore_barrier()
        pltpu.sync_copy(acc_sh.at[pl.ds(s * SLICE, SLICE), :],
                        out_hbm.at[c, pl.ds(s * SLICE, SLICE), :])

    return k(g1, src2d, dst2d, zslice16)


def _sc_aggregate2(g2lo, g2hi, src2d, dst2d, zslice16):
    @functools.partial(
        pl.kernel,
        out_type=jax.ShapeDtypeStruct((2, NPAD, F1), jnp.float32),
        mesh=_mesh(),
        compiler_params=_SC_PARAMS,
        scratch_types=[
            pltpu.VMEM((5, ROW), jnp.int32),
            pltpu.VMEM((5, ROW), jnp.int32),
            pltpu.VMEM((5, ROW, F1), jnp.float32),
            pltpu.VMEM_SHARED((NPAD, F1), jnp.float32),
            pltpu.SemaphoreType.DMA,
        ],
    )
    def k(glo_hbm, ghi_hbm, src_hbm, dst_hbm, z_hbm, out_hbm,
          si_v, di_v, rows_v, acc_sh, sem):
        c = lax.axis_index("c")
        s = lax.axis_index("s")
        pltpu.sync_copy(z_hbm, acc_sh.at[pl.ds(s * SLICE, SLICE), :])
        plsc.subcore_barrier()
        base = s * (NROWS // NTEC)

        def outer(t, carry):
            r0 = base + t * 5
            pltpu.sync_copy(src_hbm.at[pl.ds(r0, 5), :], si_v)
            pltpu.sync_copy(dst_hbm.at[pl.ds(r0, 5), :], di_v)

            @pl.when(c == 0)
            def _():
                cps = [pltpu.async_copy(glo_hbm.at[si_v.at[j]],
                                        rows_v.at[j], sem) for j in range(5)]
                for j in range(5):
                    cps[j].wait()
                    pltpu.sync_copy(rows_v.at[j], acc_sh.at[di_v.at[j]],
                                    add=True)

            @pl.when(c == 1)
            def _():
                cps = [pltpu.async_copy(ghi_hbm.at[si_v.at[j]],
                                        rows_v.at[j], sem) for j in range(5)]
                for j in range(5):
                    cps[j].wait()
                    pltpu.sync_copy(rows_v.at[j], acc_sh.at[di_v.at[j]],
                                    add=True)

            return carry

        lax.fori_loop(0, (NROWS // NTEC) // 5, outer, 0)
        plsc.subcore_barrier()
        pltpu.sync_copy(acc_sh.at[pl.ds(s * SLICE, SLICE), :],
                        out_hbm.at[c, pl.ds(s * SLICE, SLICE), :])

    return k(g2lo, g2hi, src2d, dst2d, zslice16)


_HI = jax.lax.Precision.HIGHEST


def _tc_dense1(cnt2, x, W1, b1):
    def body(cnt_ref, x_ref, w_ref, b_ref, dinv_ref, g_ref, self_ref):
        deg = cnt_ref[0, :, 0:1] + cnt_ref[1, :, 0:1] + 2.0
        dinv = lax.rsqrt(deg)
        h1 = jnp.dot(x_ref[...], w_ref[...], precision=_HI)
        dinv_ref[...] = dinv
        g_ref[...] = h1 * dinv
        self_ref[...] = (2.0 * dinv * dinv) * h1 + b_ref[...]

    return pl.pallas_call(
        body,
        grid=(GRID,),
        in_specs=[
            pl.BlockSpec((2, BLK, F1), lambda i: (0, i, 0)),
            pl.BlockSpec((BLK, 3), lambda i: (i, 0)),
            pl.BlockSpec((3, F1), lambda i: (0, 0)),
            pl.BlockSpec((1, F1), lambda i: (0, 0)),
        ],
        out_specs=[
            pl.BlockSpec((BLK, 1), lambda i: (i, 0)),
            pl.BlockSpec((BLK, F1), lambda i: (i, 0)),
            pl.BlockSpec((BLK, F1), lambda i: (i, 0)),
        ],
        out_shape=[
            jax.ShapeDtypeStruct((N, 1), jnp.float32),
            jax.ShapeDtypeStruct((N, F1), jnp.float32),
            jax.ShapeDtypeStruct((N, F1), jnp.float32),
        ],
    )(cnt2, x, W1, b1)


def _tc_dense2(acc1, self1, dinv, W2, b2):
    def body(a_ref, s_ref, d_ref, w_ref, b_ref, glo_ref, ghi_ref, self_ref):
        a = a_ref[0] + a_ref[1]
        dinv = d_ref[...]
        x2 = jnp.maximum(dinv * a + s_ref[...], 0.0)
        h2 = jnp.dot(x2, w_ref[...], precision=_HI)
        g2 = h2 * dinv
        glo_ref[...] = g2[:, :F1]
        ghi_ref[...] = g2[:, F1:]
        self_ref[...] = (2.0 * dinv * dinv) * h2 + b_ref[...]

    return pl.pallas_call(
        body,
        grid=(GRID,),
        in_specs=[
            pl.BlockSpec((2, BLK, F1), lambda i: (0, i, 0)),
            pl.BlockSpec((BLK, F1), lambda i: (i, 0)),
            pl.BlockSpec((BLK, 1), lambda i: (i, 0)),
            pl.BlockSpec((F1, F2), lambda i: (0, 0)),
            pl.BlockSpec((1, F2), lambda i: (0, 0)),
        ],
        out_specs=[
            pl.BlockSpec((BLK, F1), lambda i: (i, 0)),
            pl.BlockSpec((BLK, F1), lambda i: (i, 0)),
            pl.BlockSpec((BLK, F2), lambda i: (i, 0)),
        ],
        out_shape=[
            jax.ShapeDtypeStruct((N, F1), jnp.float32),
            jax.ShapeDtypeStruct((N, F1), jnp.float32),
            jax.ShapeDtypeStruct((N, F2), jnp.float32),
        ],
    )(acc1, self1, dinv, W2, b2)


def _tc_dense3(acc2, self2, dinv, batch2d, W3, b3):
    def body(a_ref, s_ref, d_ref, b_ref, w_ref, bias_ref, out_ref,
             pooled, cnts):
        i = pl.program_id(0)

        @pl.when(i == 0)
        def _():
            pooled[...] = jnp.zeros_like(pooled)
            cnts[...] = jnp.zeros_like(cnts)

        out2 = jnp.maximum(
            d_ref[...] * jnp.concatenate([a_ref[0], a_ref[1]], axis=1)
            + s_ref[...], 0.0)
        seg = b_ref[...]
        iota = lax.broadcasted_iota(jnp.int32, (BLK, NSEG), 1)
        oh = (seg == iota).astype(jnp.float32)
        pooled[...] += lax.dot_general(oh, out2, (((0,), (0,)), ((), ())),
                                       precision=_HI)
        cnts[...] += lax.dot_general(oh, jnp.ones((BLK, 1), jnp.float32),
                                     (((0,), (0,)), ((), ())), precision=_HI)

        @pl.when(i == GRID - 1)
        def _():
            mean = pooled[...] / jnp.maximum(cnts[...], 1.0)
            out_ref[...] = jnp.dot(mean, w_ref[...], precision=_HI) \
                + bias_ref[...]

    return pl.pallas_call(
        body,
        grid=(GRID,),
        in_specs=[
            pl.BlockSpec((2, BLK, F1), lambda i: (0, i, 0)),
            pl.BlockSpec((BLK, F2), lambda i: (i, 0)),
            pl.BlockSpec((BLK, 1), lambda i: (i, 0)),
            pl.BlockSpec((BLK, 1), lambda i: (i, 0)),
            pl.BlockSpec((F2, F2), lambda i: (0, 0)),
            pl.BlockSpec((1, F2), lambda i: (0, 0)),
        ],
        out_specs=pl.BlockSpec((NSEG, F2), lambda i: (0, 0)),
        out_shape=jax.ShapeDtypeStruct((NSEG, F2), jnp.float32),
        scratch_shapes=[
            pltpu.VMEM((NSEG, F2), jnp.float32),
            pltpu.VMEM((NSEG, 1), jnp.float32),
        ],
    )(acc2, self2, dinv, batch2d, W3, b3)


def kernel(x, edge_index, batch, W1, b1, W2, b2, W3, b3):
    src2d = edge_index[0].reshape(NROWS, ROW)
    dst2d = edge_index[1].reshape(NROWS, ROW)
    ones_row = jnp.ones((ROW, F1), jnp.float32)
    zslice1 = jnp.zeros((SLICE, F1), jnp.float32)
    zslice16 = jnp.zeros((SLICE, F1), jnp.float32)

    cnt2 = _sc_count(dst2d, ones_row, zslice1)
    dinv, g1, self1 = _tc_dense1(cnt2, x, W1, b1.reshape(1, F1))
    acc1 = _sc_aggregate1(g1, src2d, dst2d, zslice16)
    g2lo, g2hi, self2 = _tc_dense2(acc1, self1, dinv, W2, b2.reshape(1, F2))
    acc2 = _sc_aggregate2(g2lo, g2hi, src2d, dst2d, zslice16)
    out = _tc_dense3(acc2, self2, dinv, batch.reshape(N, 1), W3,
                     b3.reshape(1, F2))
    return out


# speedup vs baseline: 23.8315x; 1.0709x over previous
import functools

import jax
import jax.numpy as jnp
from jax import lax
from jax.experimental import pallas as pl
from jax.experimental.pallas import tpu as pltpu
from jax.experimental.pallas import tpu_sc as plsc

N = 100000
E = 1600000
NSEG = 256
F1 = 16
F2 = 32

NTEC = 16
NPAD = 100096
SLICE = NPAD // NTEC
ROW = 80
NROWS = E // ROW

BLK = 2000
GRID = N // BLK


def _mesh():
    return plsc.VectorSubcoreMesh(core_axis_name="c", subcore_axis_name="s")


_SC_PARAMS = pltpu.CompilerParams(use_tc_tiling_on_sc=False)


def _sc_count(dst2d, ones_row, zslice1):
    @functools.partial(
        pl.kernel,
        out_type=jax.ShapeDtypeStruct((2, NPAD, F1), jnp.float32),
        mesh=_mesh(),
        compiler_params=_SC_PARAMS,
        scratch_types=[
            pltpu.VMEM((5, ROW), jnp.int32),
            pltpu.VMEM((ROW, F1), jnp.float32),
            pltpu.VMEM_SHARED((NPAD, F1), jnp.float32),
            pltpu.SemaphoreType.DMA,
        ],
    )
    def k(dst_hbm, ones_hbm, z_hbm, out_hbm, di_v, ones_v, cnt_sh, sem2):
        c = lax.axis_index("c")
        s = lax.axis_index("s")
        pltpu.sync_copy(ones_hbm, ones_v)
        pltpu.sync_copy(z_hbm, cnt_sh.at[pl.ds(s * SLICE, SLICE), :])
        plsc.subcore_barrier()
        base = (c * NTEC + s) * (NROWS // 32)

        def outer(t, carry):
            pltpu.sync_copy(dst_hbm.at[pl.ds(base + t * 5, 5), :], di_v)
            scs = [pltpu.async_copy(ones_v, cnt_sh.at[di_v.at[j]], sem2,
                                    add=True) for j in range(5)]
            for sc in scs:
                sc.wait()
            return carry

        lax.fori_loop(0, (NROWS // 32) // 5, outer, 0)
        plsc.subcore_barrier()
        pltpu.sync_copy(cnt_sh.at[pl.ds(s * SLICE, SLICE), :],
                        out_hbm.at[c, pl.ds(s * SLICE, SLICE), :])

    return k(dst2d, ones_row, zslice1)


def _sc_aggregate1(g1, src2d, dst2d, zslice16):
    @functools.partial(
        pl.kernel,
        out_type=jax.ShapeDtypeStruct((2, NPAD, F1), jnp.float32),
        mesh=_mesh(),
        compiler_params=_SC_PARAMS,
        scratch_types=[
            pltpu.VMEM((5, ROW), jnp.int32),
            pltpu.VMEM((5, ROW), jnp.int32),
            pltpu.VMEM((5, ROW, F1), jnp.float32),
            pltpu.VMEM_SHARED((NPAD, F1), jnp.float32),
            pltpu.SemaphoreType.DMA,
            pltpu.SemaphoreType.DMA,
        ],
    )
    def k(g_hbm, src_hbm, dst_hbm, z_hbm, out_hbm,
          si_v, di_v, rows_v, acc_sh, sem, sem2):
        c = lax.axis_index("c")
        s = lax.axis_index("s")
        pltpu.sync_copy(z_hbm, acc_sh.at[pl.ds(s * SLICE, SLICE), :])
        plsc.subcore_barrier()
        base = (c * NTEC + s) * (NROWS // 32)

        def outer(t, carry):
            r0 = base + t * 5
            pltpu.sync_copy(src_hbm.at[pl.ds(r0, 5), :], si_v)
            pltpu.sync_copy(dst_hbm.at[pl.ds(r0, 5), :], di_v)
            cps = [pltpu.async_copy(g_hbm.at[si_v.at[j]], rows_v.at[j], sem)
                   for j in range(5)]
            scs = []
            for j in range(5):
                cps[j].wait()
                scs.append(pltpu.async_copy(rows_v.at[j],
                                            acc_sh.at[di_v.at[j]], sem2,
                                            add=True))
            for sc in scs:
                sc.wait()
            return carry

        lax.fori_loop(0, (NROWS // 32) // 5, outer, 0)
        plsc.subcore_barrier()
        pltpu.sync_copy(acc_sh.at[pl.ds(s * SLICE, SLICE), :],
                        out_hbm.at[c, pl.ds(s * SLICE, SLICE), :])

    return k(g1, src2d, dst2d, zslice16)


def _sc_aggregate2(g2lo, g2hi, src2d, dst2d, zslice16):
    @functools.partial(
        pl.kernel,
        out_type=jax.ShapeDtypeStruct((2, NPAD, F1), jnp.float32),
        mesh=_mesh(),
        compiler_params=_SC_PARAMS,
        scratch_types=[
            pltpu.VMEM((5, ROW), jnp.int32),
            pltpu.VMEM((5, ROW), jnp.int32),
            pltpu.VMEM((5, ROW, F1), jnp.float32),
            pltpu.VMEM_SHARED((NPAD, F1), jnp.float32),
            pltpu.SemaphoreType.DMA,
            pltpu.SemaphoreType.DMA,
        ],
    )
    def k(glo_hbm, ghi_hbm, src_hbm, dst_hbm, z_hbm, out_hbm,
          si_v, di_v, rows_v, acc_sh, sem, sem2):
        c = lax.axis_index("c")
        s = lax.axis_index("s")
        pltpu.sync_copy(z_hbm, acc_sh.at[pl.ds(s * SLICE, SLICE), :])
        plsc.subcore_barrier()
        base = s * (NROWS // NTEC)

        def outer(t, carry):
            r0 = base + t * 5
            pltpu.sync_copy(src_hbm.at[pl.ds(r0, 5), :], si_v)
            pltpu.sync_copy(dst_hbm.at[pl.ds(r0, 5), :], di_v)

            @pl.when(c == 0)
            def _():
                cps = [pltpu.async_copy(glo_hbm.at[si_v.at[j]],
                                        rows_v.at[j], sem) for j in range(5)]
                scs = []
                for j in range(5):
                    cps[j].wait()
                    scs.append(pltpu.async_copy(rows_v.at[j],
                                                acc_sh.at[di_v.at[j]], sem2,
                                                add=True))
                for sc in scs:
                    sc.wait()

            @pl.when(c == 1)
            def _():
                cps = [pltpu.async_copy(ghi_hbm.at[si_v.at[j]],
                                        rows_v.at[j], sem) for j in range(5)]
                scs = []
                for j in range(5):
                    cps[j].wait()
                    scs.append(pltpu.async_copy(rows_v.at[j],
                                                acc_sh.at[di_v.at[j]], sem2,
                                                add=True))
                for sc in scs:
                    sc.wait()

            return carry

        lax.fori_loop(0, (NROWS // NTEC) // 5, outer, 0)
        plsc.subcore_barrier()
        pltpu.sync_copy(acc_sh.at[pl.ds(s * SLICE, SLICE), :],
                        out_hbm.at[c, pl.ds(s * SLICE, SLICE), :])

    return k(g2lo, g2hi, src2d, dst2d, zslice16)


_HI = jax.lax.Precision.HIGHEST


def _tc_dense1(cnt2, x, W1, b1):
    def body(cnt_ref, x_ref, w_ref, b_ref, dinv_ref, g_ref, self_ref):
        deg = cnt_ref[0, :, 0:1] + cnt_ref[1, :, 0:1] + 2.0
        dinv = lax.rsqrt(deg)
        h1 = jnp.dot(x_ref[...], w_ref[...], precision=_HI)
        dinv_ref[...] = dinv
        g_ref[...] = h1 * dinv
        self_ref[...] = (2.0 * dinv * dinv) * h1 + b_ref[...]

    return pl.pallas_call(
        body,
        grid=(GRID,),
        in_specs=[
            pl.BlockSpec((2, BLK, F1), lambda i: (0, i, 0)),
            pl.BlockSpec((BLK, 3), lambda i: (i, 0)),
            pl.BlockSpec((3, F1), lambda i: (0, 0)),
            pl.BlockSpec((1, F1), lambda i: (0, 0)),
        ],
        out_specs=[
            pl.BlockSpec((BLK, 1), lambda i: (i, 0)),
            pl.BlockSpec((BLK, F1), lambda i: (i, 0)),
            pl.BlockSpec((BLK, F1), lambda i: (i, 0)),
        ],
        out_shape=[
            jax.ShapeDtypeStruct((N, 1), jnp.float32),
            jax.ShapeDtypeStruct((N, F1), jnp.float32),
            jax.ShapeDtypeStruct((N, F1), jnp.float32),
        ],
    )(cnt2, x, W1, b1)


def _tc_dense2(acc1, self1, dinv, W2, b2):
    def body(a_ref, s_ref, d_ref, w_ref, b_ref, glo_ref, ghi_ref, self_ref):
        a = a_ref[0] + a_ref[1]
        dinv = d_ref[...]
        x2 = jnp.maximum(dinv * a + s_ref[...], 0.0)
        h2 = jnp.dot(x2, w_ref[...], precision=_HI)
        g2 = h2 * dinv
        glo_ref[...] = g2[:, :F1]
        ghi_ref[...] = g2[:, F1:]
        self_ref[...] = (2.0 * dinv * dinv) * h2 + b_ref[...]

    return pl.pallas_call(
        body,
        grid=(GRID,),
        in_specs=[
            pl.BlockSpec((2, BLK, F1), lambda i: (0, i, 0)),
            pl.BlockSpec((BLK, F1), lambda i: (i, 0)),
            pl.BlockSpec((BLK, 1), lambda i: (i, 0)),
            pl.BlockSpec((F1, F2), lambda i: (0, 0)),
            pl.BlockSpec((1, F2), lambda i: (0, 0)),
        ],
        out_specs=[
            pl.BlockSpec((BLK, F1), lambda i: (i, 0)),
            pl.BlockSpec((BLK, F1), lambda i: (i, 0)),
            pl.BlockSpec((BLK, F2), lambda i: (i, 0)),
        ],
        out_shape=[
            jax.ShapeDtypeStruct((N, F1), jnp.float32),
            jax.ShapeDtypeStruct((N, F1), jnp.float32),
            jax.ShapeDtypeStruct((N, F2), jnp.float32),
        ],
    )(acc1, self1, dinv, W2, b2)


def _tc_dense3(acc2, self2, dinv, batch2d, W3, b3):
    def body(a_ref, s_ref, d_ref, b_ref, w_ref, bias_ref, out_ref,
             pooled, cnts):
        i = pl.program_id(0)

        @pl.when(i == 0)
        def _():
            pooled[...] = jnp.zeros_like(pooled)
            cnts[...] = jnp.zeros_like(cnts)

        out2 = jnp.maximum(
            d_ref[...] * jnp.concatenate([a_ref[0], a_ref[1]], axis=1)
            + s_ref[...], 0.0)
        seg = b_ref[...]
        iota = lax.broadcasted_iota(jnp.int32, (BLK, NSEG), 1)
        oh = (seg == iota).astype(jnp.float32)
        pooled[...] += lax.dot_general(oh, out2, (((0,), (0,)), ((), ())),
                                       precision=_HI)
        cnts[...] += lax.dot_general(oh, jnp.ones((BLK, 1), jnp.float32),
                                     (((0,), (0,)), ((), ())), precision=_HI)

        @pl.when(i == GRID - 1)
        def _():
            mean = pooled[...] / jnp.maximum(cnts[...], 1.0)
            out_ref[...] = jnp.dot(mean, w_ref[...], precision=_HI) \
                + bias_ref[...]

    return pl.pallas_call(
        body,
        grid=(GRID,),
        in_specs=[
            pl.BlockSpec((2, BLK, F1), lambda i: (0, i, 0)),
            pl.BlockSpec((BLK, F2), lambda i: (i, 0)),
            pl.BlockSpec((BLK, 1), lambda i: (i, 0)),
            pl.BlockSpec((BLK, 1), lambda i: (i, 0)),
            pl.BlockSpec((F2, F2), lambda i: (0, 0)),
            pl.BlockSpec((1, F2), lambda i: (0, 0)),
        ],
        out_specs=pl.BlockSpec((NSEG, F2), lambda i: (0, 0)),
        out_shape=jax.ShapeDtypeStruct((NSEG, F2), jnp.float32),
        scratch_shapes=[
            pltpu.VMEM((NSEG, F2), jnp.float32),
            pltpu.VMEM((NSEG, 1), jnp.float32),
        ],
    )(acc2, self2, dinv, batch2d, W3, b3)


def kernel(x, edge_index, batch, W1, b1, W2, b2, W3, b3):
    src2d = edge_index[0].reshape(NROWS, ROW)
    dst2d = edge_index[1].reshape(NROWS, ROW)
    ones_row = jnp.ones((ROW, F1), jnp.float32)
    zslice1 = jnp.zeros((SLICE, F1), jnp.float32)
    zslice16 = jnp.zeros((SLICE, F1), jnp.float32)

    cnt2 = _sc_count(dst2d, ones_row, zslice1)
    dinv, g1, self1 = _tc_dense1(cnt2, x, W1, b1.reshape(1, F1))
    acc1 = _sc_aggregate1(g1, src2d, dst2d, zslice16)
    g2lo, g2hi, self2 = _tc_dense2(acc1, self1, dinv, W2, b2.reshape(1, F2))
    acc2 = _sc_aggregate2(g2lo, g2hi, src2d, dst2d, zslice16)
    out = _tc_dense3(acc2, self2, dinv, batch.reshape(N, 1), W3,
                     b3.reshape(1, F2))
    return out


# speedup vs baseline: 24.1399x; 1.0129x over previous
import functools

import jax
import jax.numpy as jnp
from jax import lax
from jax.experimental import pallas as pl
from jax.experimental.pallas import tpu as pltpu
from jax.experimental.pallas import tpu_sc as plsc

N = 100000
E = 1600000
NSEG = 256
F1 = 16
F2 = 32

NTEC = 16
NPAD = 100096
SLICE = NPAD // NTEC
ROW = 80
NROWS = E // ROW

BLK = 2000
GRID = N // BLK


def _mesh():
    return plsc.VectorSubcoreMesh(core_axis_name="c", subcore_axis_name="s")


_SC_PARAMS = pltpu.CompilerParams(use_tc_tiling_on_sc=False)


def _sc_count(dst2d, ones_row, zslice1):
    @functools.partial(
        pl.kernel,
        out_type=jax.ShapeDtypeStruct((2, NPAD, F1), jnp.bfloat16),
        mesh=_mesh(),
        compiler_params=_SC_PARAMS,
        scratch_types=[
            pltpu.VMEM((5, ROW), jnp.int32),
            pltpu.VMEM((ROW, F1), jnp.bfloat16),
            pltpu.VMEM_SHARED((NPAD, F1), jnp.bfloat16),
            pltpu.SemaphoreType.DMA,
        ],
    )
    def k(dst_hbm, ones_hbm, z_hbm, out_hbm, di_v, ones_v, cnt_sh, sem2):
        c = lax.axis_index("c")
        s = lax.axis_index("s")
        pltpu.sync_copy(ones_hbm, ones_v)
        pltpu.sync_copy(z_hbm, cnt_sh.at[pl.ds(s * SLICE, SLICE), :])
        plsc.subcore_barrier()
        base = (c * NTEC + s) * (NROWS // 32)

        def outer(t, carry):
            pltpu.sync_copy(dst_hbm.at[pl.ds(base + t * 5, 5), :], di_v)
            scs = [pltpu.async_copy(ones_v, cnt_sh.at[di_v.at[j]], sem2,
                                    add=True) for j in range(5)]
            for sc in scs:
                sc.wait()
            return carry

        lax.fori_loop(0, (NROWS // 32) // 5, outer, 0)
        plsc.subcore_barrier()
        pltpu.sync_copy(cnt_sh.at[pl.ds(s * SLICE, SLICE), :],
                        out_hbm.at[c, pl.ds(s * SLICE, SLICE), :])

    return k(dst2d, ones_row, zslice1)


def _sc_aggregate1(g1, src2d, dst2d, zslice16):
    @functools.partial(
        pl.kernel,
        out_type=jax.ShapeDtypeStruct((2, NPAD, F1), jnp.bfloat16),
        mesh=_mesh(),
        compiler_params=_SC_PARAMS,
        scratch_types=[
            pltpu.VMEM((5, ROW), jnp.int32),
            pltpu.VMEM((5, ROW), jnp.int32),
            pltpu.VMEM((5, ROW, F1), jnp.bfloat16),
            pltpu.VMEM_SHARED((NPAD, F1), jnp.bfloat16),
            pltpu.SemaphoreType.DMA,
            pltpu.SemaphoreType.DMA,
        ],
    )
    def k(g_hbm, src_hbm, dst_hbm, z_hbm, out_hbm,
          si_v, di_v, rows_v, acc_sh, sem, sem2):
        c = lax.axis_index("c")
        s = lax.axis_index("s")
        pltpu.sync_copy(z_hbm, acc_sh.at[pl.ds(s * SLICE, SLICE), :])
        plsc.subcore_barrier()
        base = (c * NTEC + s) * (NROWS // 32)

        def outer(t, carry):
            r0 = base + t * 5
            pltpu.sync_copy(src_hbm.at[pl.ds(r0, 5), :], si_v)
            pltpu.sync_copy(dst_hbm.at[pl.ds(r0, 5), :], di_v)
            cps = [pltpu.async_copy(g_hbm.at[si_v.at[j]], rows_v.at[j], sem)
                   for j in range(5)]
            scs = []
            for j in range(5):
                cps[j].wait()
                scs.append(pltpu.async_copy(rows_v.at[j],
                                            acc_sh.at[di_v.at[j]], sem2,
                                            add=True))
            for sc in scs:
                sc.wait()
            return carry

        lax.fori_loop(0, (NROWS // 32) // 5, outer, 0)
        plsc.subcore_barrier()
        pltpu.sync_copy(acc_sh.at[pl.ds(s * SLICE, SLICE), :],
                        out_hbm.at[c, pl.ds(s * SLICE, SLICE), :])

    return k(g1, src2d, dst2d, zslice16)


def _sc_aggregate2(g2lo, g2hi, src2d, dst2d, zslice16):
    @functools.partial(
        pl.kernel,
        out_type=jax.ShapeDtypeStruct((2, NPAD, F1), jnp.bfloat16),
        mesh=_mesh(),
        compiler_params=_SC_PARAMS,
        scratch_types=[
            pltpu.VMEM((5, ROW), jnp.int32),
            pltpu.VMEM((5, ROW), jnp.int32),
            pltpu.VMEM((5, ROW, F1), jnp.bfloat16),
            pltpu.VMEM_SHARED((NPAD, F1), jnp.bfloat16),
            pltpu.SemaphoreType.DMA,
            pltpu.SemaphoreType.DMA,
        ],
    )
    def k(glo_hbm, ghi_hbm, src_hbm, dst_hbm, z_hbm, out_hbm,
          si_v, di_v, rows_v, acc_sh, sem, sem2):
        c = lax.axis_index("c")
        s = lax.axis_index("s")
        pltpu.sync_copy(z_hbm, acc_sh.at[pl.ds(s * SLICE, SLICE), :])
        plsc.subcore_barrier()
        base = s * (NROWS // NTEC)

        def outer(t, carry):
            r0 = base + t * 5
            pltpu.sync_copy(src_hbm.at[pl.ds(r0, 5), :], si_v)
            pltpu.sync_copy(dst_hbm.at[pl.ds(r0, 5), :], di_v)

            @pl.when(c == 0)
            def _():
                cps = [pltpu.async_copy(glo_hbm.at[si_v.at[j]],
                                        rows_v.at[j], sem) for j in range(5)]
                scs = []
                for j in range(5):
                    cps[j].wait()
                    scs.append(pltpu.async_copy(rows_v.at[j],
                                                acc_sh.at[di_v.at[j]], sem2,
                                                add=True))
                for sc in scs:
                    sc.wait()

            @pl.when(c == 1)
            def _():
                cps = [pltpu.async_copy(ghi_hbm.at[si_v.at[j]],
                                        rows_v.at[j], sem) for j in range(5)]
                scs = []
                for j in range(5):
                    cps[j].wait()
                    scs.append(pltpu.async_copy(rows_v.at[j],
                                                acc_sh.at[di_v.at[j]], sem2,
                                                add=True))
                for sc in scs:
                    sc.wait()

            return carry

        lax.fori_loop(0, (NROWS // NTEC) // 5, outer, 0)
        plsc.subcore_barrier()
        pltpu.sync_copy(acc_sh.at[pl.ds(s * SLICE, SLICE), :],
                        out_hbm.at[c, pl.ds(s * SLICE, SLICE), :])

    return k(g2lo, g2hi, src2d, dst2d, zslice16)


_HI = jax.lax.Precision.HIGHEST


def _tc_dense1(cnt2, x, W1, b1):
    def body(cnt_ref, x_ref, w_ref, b_ref, dinv_ref, g_ref, self_ref):
        deg = (cnt_ref[0, :, 0:1].astype(jnp.float32)
               + cnt_ref[1, :, 0:1].astype(jnp.float32) + 2.0)
        dinv = lax.rsqrt(deg)
        h1 = jnp.dot(x_ref[...], w_ref[...], precision=_HI)
        dinv_ref[...] = dinv
        g_ref[...] = (h1 * dinv).astype(jnp.bfloat16)
        self_ref[...] = (2.0 * dinv * dinv) * h1 + b_ref[...]

    return pl.pallas_call(
        body,
        grid=(GRID,),
        in_specs=[
            pl.BlockSpec((2, BLK, F1), lambda i: (0, i, 0)),
            pl.BlockSpec((BLK, 3), lambda i: (i, 0)),
            pl.BlockSpec((3, F1), lambda i: (0, 0)),
            pl.BlockSpec((1, F1), lambda i: (0, 0)),
        ],
        out_specs=[
            pl.BlockSpec((BLK, 1), lambda i: (i, 0)),
            pl.BlockSpec((BLK, F1), lambda i: (i, 0)),
            pl.BlockSpec((BLK, F1), lambda i: (i, 0)),
        ],
        out_shape=[
            jax.ShapeDtypeStruct((N, 1), jnp.float32),
            jax.ShapeDtypeStruct((N, F1), jnp.bfloat16),
            jax.ShapeDtypeStruct((N, F1), jnp.float32),
        ],
    )(cnt2, x, W1, b1)


def _tc_dense2(acc1, self1, dinv, W2, b2):
    def body(a_ref, s_ref, d_ref, w_ref, b_ref, glo_ref, ghi_ref, self_ref):
        a = a_ref[0].astype(jnp.float32) \
            + a_ref[1].astype(jnp.float32)
        dinv = d_ref[...]
        x2 = jnp.maximum(dinv * a + s_ref[...], 0.0)
        h2 = jnp.dot(x2, w_ref[...], precision=_HI)
        g2 = (h2 * dinv).astype(jnp.bfloat16)
        glo_ref[...] = g2[:, :F1]
        ghi_ref[...] = g2[:, F1:]
        self_ref[...] = (2.0 * dinv * dinv) * h2 + b_ref[...]

    return pl.pallas_call(
        body,
        grid=(GRID,),
        in_specs=[
            pl.BlockSpec((2, BLK, F1), lambda i: (0, i, 0)),
            pl.BlockSpec((BLK, F1), lambda i: (i, 0)),
            pl.BlockSpec((BLK, 1), lambda i: (i, 0)),
            pl.BlockSpec((F1, F2), lambda i: (0, 0)),
            pl.BlockSpec((1, F2), lambda i: (0, 0)),
        ],
        out_specs=[
            pl.BlockSpec((BLK, F1), lambda i: (i, 0)),
            pl.BlockSpec((BLK, F1), lambda i: (i, 0)),
            pl.BlockSpec((BLK, F2), lambda i: (i, 0)),
        ],
        out_shape=[
            jax.ShapeDtypeStruct((N, F1), jnp.bfloat16),
            jax.ShapeDtypeStruct((N, F1), jnp.bfloat16),
            jax.ShapeDtypeStruct((N, F2), jnp.float32),
        ],
    )(acc1, self1, dinv, W2, b2)


def _tc_dense3(acc2, self2, dinv, batch2d, W3, b3):
    def body(a_ref, s_ref, d_ref, b_ref, w_ref, bias_ref, out_ref,
             pooled, cnts):
        i = pl.program_id(0)

        @pl.when(i == 0)
        def _():
            pooled[...] = jnp.zeros_like(pooled)
            cnts[...] = jnp.zeros_like(cnts)

        out2 = jnp.maximum(
            d_ref[...] * jnp.concatenate(
                [a_ref[0].astype(jnp.float32), a_ref[1].astype(jnp.float32)],
                axis=1)
            + s_ref[...], 0.0)
        seg = b_ref[...]
        iota = lax.broadcasted_iota(jnp.int32, (BLK, NSEG), 1)
        oh = (seg == iota).astype(jnp.float32)
        pooled[...] += lax.dot_general(oh, out2, (((0,), (0,)), ((), ())),
                                       precision=_HI)
        cnts[...] += lax.dot_general(oh, jnp.ones((BLK, 1), jnp.float32),
                                     (((0,), (0,)), ((), ())), precision=_HI)

        @pl.when(i == GRID - 1)
        def _():
            mean = pooled[...] / jnp.maximum(cnts[...], 1.0)
            out_ref[...] = jnp.dot(mean, w_ref[...], precision=_HI) \
                + bias_ref[...]

    return pl.pallas_call(
        body,
        grid=(GRID,),
        in_specs=[
            pl.BlockSpec((2, BLK, F1), lambda i: (0, i, 0)),
            pl.BlockSpec((BLK, F2), lambda i: (i, 0)),
            pl.BlockSpec((BLK, 1), lambda i: (i, 0)),
            pl.BlockSpec((BLK, 1), lambda i: (i, 0)),
            pl.BlockSpec((F2, F2), lambda i: (0, 0)),
            pl.BlockSpec((1, F2), lambda i: (0, 0)),
        ],
        out_specs=pl.BlockSpec((NSEG, F2), lambda i: (0, 0)),
        out_shape=jax.ShapeDtypeStruct((NSEG, F2), jnp.float32),
        scratch_shapes=[
            pltpu.VMEM((NSEG, F2), jnp.float32),
            pltpu.VMEM((NSEG, 1), jnp.float32),
        ],
    )(acc2, self2, dinv, batch2d, W3, b3)


def kernel(x, edge_index, batch, W1, b1, W2, b2, W3, b3):
    src2d = edge_index[0].reshape(NROWS, ROW)
    dst2d = edge_index[1].reshape(NROWS, ROW)
    ones_row = jnp.ones((ROW, F1), jnp.bfloat16)
    zslice1 = jnp.zeros((SLICE, F1), jnp.bfloat16)
    zslice16 = zslice1

    cnt2 = _sc_count(dst2d, ones_row, zslice1)
    dinv, g1, self1 = _tc_dense1(cnt2, x, W1, b1.reshape(1, F1))
    acc1 = _sc_aggregate1(g1, src2d, dst2d, zslice16)
    g2lo, g2hi, self2 = _tc_dense2(acc1, self1, dinv, W2, b2.reshape(1, F2))
    acc2 = _sc_aggregate2(g2lo, g2hi, src2d, dst2d, zslice16)
    out = _tc_dense3(acc2, self2, dinv, batch.reshape(N, 1), W3,
                     b3.reshape(1, F2))
    return out


# speedup vs baseline: 25.1864x; 1.0434x over previous
import functools

import jax
import jax.numpy as jnp
from jax import lax
from jax.experimental import pallas as pl
from jax.experimental.pallas import tpu as pltpu
from jax.experimental.pallas import tpu_sc as plsc

N = 100000
E = 1600000
NSEG = 256
F1 = 16
F2 = 32

NTEC = 16
NPAD = 100096
SLICE = NPAD // NTEC
ROW = 80
NROWS = E // ROW

BLK = 4000
GRID = N // BLK


def _mesh():
    return plsc.VectorSubcoreMesh(core_axis_name="c", subcore_axis_name="s")


_SC_PARAMS = pltpu.CompilerParams(use_tc_tiling_on_sc=False)


def _sc_count(ei3, ones_row, zslice1):
    @functools.partial(
        pl.kernel,
        out_type=jax.ShapeDtypeStruct((2, NPAD, F1), jnp.bfloat16),
        mesh=_mesh(),
        compiler_params=_SC_PARAMS,
        scratch_types=[
            pltpu.VMEM((5, ROW), jnp.int32),
            pltpu.VMEM((ROW, F1), jnp.bfloat16),
            pltpu.VMEM_SHARED((NPAD, F1), jnp.bfloat16),
            pltpu.SemaphoreType.DMA,
        ],
    )
    def k(ei_hbm, ones_hbm, z_hbm, out_hbm, di_v, ones_v, cnt_sh, sem2):
        c = lax.axis_index("c")
        s = lax.axis_index("s")
        pltpu.sync_copy(ones_hbm, ones_v)
        pltpu.sync_copy(z_hbm, cnt_sh.at[pl.ds(s * SLICE, SLICE), :])
        plsc.subcore_barrier()
        base = (c * NTEC + s) * (NROWS // 32)

        def outer(t, carry):
            pltpu.sync_copy(ei_hbm.at[1, pl.ds(base + t * 5, 5), :], di_v)
            scs = [pltpu.async_copy(ones_v, cnt_sh.at[di_v.at[j]], sem2,
                                    add=True) for j in range(5)]
            for sc in scs:
                sc.wait()
            return carry

        lax.fori_loop(0, (NROWS // 32) // 5, outer, 0)
        plsc.subcore_barrier()
        pltpu.sync_copy(cnt_sh.at[pl.ds(s * SLICE, SLICE), :],
                        out_hbm.at[c, pl.ds(s * SLICE, SLICE), :])

    return k(ei3, ones_row, zslice1)


def _sc_aggregate1(g1, ei3, zslice16):
    @functools.partial(
        pl.kernel,
        out_type=jax.ShapeDtypeStruct((2, NPAD, F1), jnp.bfloat16),
        mesh=_mesh(),
        compiler_params=_SC_PARAMS,
        scratch_types=[
            pltpu.VMEM((5, ROW), jnp.int32),
            pltpu.VMEM((5, ROW), jnp.int32),
            pltpu.VMEM((5, ROW, F1), jnp.bfloat16),
            pltpu.VMEM_SHARED((NPAD, F1), jnp.bfloat16),
            pltpu.SemaphoreType.DMA,
            pltpu.SemaphoreType.DMA,
        ],
    )
    def k(g_hbm, ei_hbm, z_hbm, out_hbm,
          si_v, di_v, rows_v, acc_sh, sem, sem2):
        c = lax.axis_index("c")
        s = lax.axis_index("s")
        pltpu.sync_copy(z_hbm, acc_sh.at[pl.ds(s * SLICE, SLICE), :])
        plsc.subcore_barrier()
        base = (c * NTEC + s) * (NROWS // 32)

        def outer(t, carry):
            r0 = base + t * 5
            pltpu.sync_copy(ei_hbm.at[0, pl.ds(r0, 5), :], si_v)
            pltpu.sync_copy(ei_hbm.at[1, pl.ds(r0, 5), :], di_v)
            cps = [pltpu.async_copy(g_hbm.at[si_v.at[j]], rows_v.at[j], sem)
                   for j in range(5)]
            scs = []
            for j in range(5):
                cps[j].wait()
                scs.append(pltpu.async_copy(rows_v.at[j],
                                            acc_sh.at[di_v.at[j]], sem2,
                                            add=True))
            for sc in scs:
                sc.wait()
            return carry

        lax.fori_loop(0, (NROWS // 32) // 5, outer, 0)
        plsc.subcore_barrier()
        pltpu.sync_copy(acc_sh.at[pl.ds(s * SLICE, SLICE), :],
                        out_hbm.at[c, pl.ds(s * SLICE, SLICE), :])

    return k(g1, ei3, zslice16)


def _sc_aggregate2(g2lo, g2hi, ei3, zslice16):
    @functools.partial(
        pl.kernel,
        out_type=jax.ShapeDtypeStruct((2, NPAD, F1), jnp.bfloat16),
        mesh=_mesh(),
        compiler_params=_SC_PARAMS,
        scratch_types=[
            pltpu.VMEM((5, ROW), jnp.int32),
            pltpu.VMEM((5, ROW), jnp.int32),
            pltpu.VMEM((5, ROW, F1), jnp.bfloat16),
            pltpu.VMEM_SHARED((NPAD, F1), jnp.bfloat16),
            pltpu.SemaphoreType.DMA,
            pltpu.SemaphoreType.DMA,
        ],
    )
    def k(glo_hbm, ghi_hbm, ei_hbm, z_hbm, out_hbm,
          si_v, di_v, rows_v, acc_sh, sem, sem2):
        c = lax.axis_index("c")
        s = lax.axis_index("s")
        pltpu.sync_copy(z_hbm, acc_sh.at[pl.ds(s * SLICE, SLICE), :])
        plsc.subcore_barrier()
        base = s * (NROWS // NTEC)

        def outer(t, carry):
            r0 = base + t * 5
            pltpu.sync_copy(ei_hbm.at[0, pl.ds(r0, 5), :], si_v)
            pltpu.sync_copy(ei_hbm.at[1, pl.ds(r0, 5), :], di_v)

            @pl.when(c == 0)
            def _():
                cps = [pltpu.async_copy(glo_hbm.at[si_v.at[j]],
                                        rows_v.at[j], sem) for j in range(5)]
                scs = []
                for j in range(5):
                    cps[j].wait()
                    scs.append(pltpu.async_copy(rows_v.at[j],
                                                acc_sh.at[di_v.at[j]], sem2,
                                                add=True))
                for sc in scs:
                    sc.wait()

            @pl.when(c == 1)
            def _():
                cps = [pltpu.async_copy(ghi_hbm.at[si_v.at[j]],
                                        rows_v.at[j], sem) for j in range(5)]
                scs = []
                for j in range(5):
                    cps[j].wait()
                    scs.append(pltpu.async_copy(rows_v.at[j],
                                                acc_sh.at[di_v.at[j]], sem2,
                                                add=True))
                for sc in scs:
                    sc.wait()

            return carry

        lax.fori_loop(0, (NROWS // NTEC) // 5, outer, 0)
        plsc.subcore_barrier()
        pltpu.sync_copy(acc_sh.at[pl.ds(s * SLICE, SLICE), :],
                        out_hbm.at[c, pl.ds(s * SLICE, SLICE), :])

    return k(g2lo, g2hi, ei3, zslice16)


_HI = jax.lax.Precision.HIGHEST


def _tc_dense1(cnt2, x, W1, b1):
    def body(cnt_ref, x_ref, w_ref, b_ref, dinv_ref, g_ref, self_ref):
        deg = (cnt_ref[0, :, 0:1].astype(jnp.float32)
               + cnt_ref[1, :, 0:1].astype(jnp.float32) + 2.0)
        dinv = lax.rsqrt(deg)
        h1 = jnp.dot(x_ref[...], w_ref[...], precision=_HI)
        dinv_ref[...] = dinv
        g_ref[...] = (h1 * dinv).astype(jnp.bfloat16)
        self_ref[...] = (2.0 * dinv * dinv) * h1 + b_ref[...]

    return pl.pallas_call(
        body,
        grid=(GRID,),
        in_specs=[
            pl.BlockSpec((2, BLK, F1), lambda i: (0, i, 0)),
            pl.BlockSpec((BLK, 3), lambda i: (i, 0)),
            pl.BlockSpec((3, F1), lambda i: (0, 0)),
            pl.BlockSpec((1, F1), lambda i: (0, 0)),
        ],
        out_specs=[
            pl.BlockSpec((BLK, 1), lambda i: (i, 0)),
            pl.BlockSpec((BLK, F1), lambda i: (i, 0)),
            pl.BlockSpec((BLK, F1), lambda i: (i, 0)),
        ],
        out_shape=[
            jax.ShapeDtypeStruct((N, 1), jnp.float32),
            jax.ShapeDtypeStruct((N, F1), jnp.bfloat16),
            jax.ShapeDtypeStruct((N, F1), jnp.float32),
        ],
    )(cnt2, x, W1, b1)


def _tc_dense2(acc1, self1, dinv, W2, b2):
    def body(a_ref, s_ref, d_ref, w_ref, b_ref, glo_ref, ghi_ref, self_ref):
        a = a_ref[0].astype(jnp.float32) \
            + a_ref[1].astype(jnp.float32)
        dinv = d_ref[...]
        x2 = jnp.maximum(dinv * a + s_ref[...], 0.0)
        h2 = jnp.dot(x2, w_ref[...], precision=_HI)
        g2 = (h2 * dinv).astype(jnp.bfloat16)
        glo_ref[...] = g2[:, :F1]
        ghi_ref[...] = g2[:, F1:]
        self_ref[...] = (2.0 * dinv * dinv) * h2 + b_ref[...]

    return pl.pallas_call(
        body,
        grid=(GRID,),
        in_specs=[
            pl.BlockSpec((2, BLK, F1), lambda i: (0, i, 0)),
            pl.BlockSpec((BLK, F1), lambda i: (i, 0)),
            pl.BlockSpec((BLK, 1), lambda i: (i, 0)),
            pl.BlockSpec((F1, F2), lambda i: (0, 0)),
            pl.BlockSpec((1, F2), lambda i: (0, 0)),
        ],
        out_specs=[
            pl.BlockSpec((BLK, F1), lambda i: (i, 0)),
            pl.BlockSpec((BLK, F1), lambda i: (i, 0)),
            pl.BlockSpec((BLK, F2), lambda i: (i, 0)),
        ],
        out_shape=[
            jax.ShapeDtypeStruct((N, F1), jnp.bfloat16),
            jax.ShapeDtypeStruct((N, F1), jnp.bfloat16),
            jax.ShapeDtypeStruct((N, F2), jnp.float32),
        ],
    )(acc1, self1, dinv, W2, b2)


def _tc_dense3(acc2, self2, dinv, batch2d, W3, b3):
    def body(a_ref, s_ref, d_ref, b_ref, w_ref, bias_ref, out_ref,
             pooled, cnts):
        i = pl.program_id(0)

        @pl.when(i == 0)
        def _():
            pooled[...] = jnp.zeros_like(pooled)
            cnts[...] = jnp.zeros_like(cnts)

        out2 = jnp.maximum(
            d_ref[...] * jnp.concatenate(
                [a_ref[0].astype(jnp.float32), a_ref[1].astype(jnp.float32)],
                axis=1)
            + s_ref[...], 0.0)
        seg = b_ref[...]
        iota = lax.broadcasted_iota(jnp.int32, (BLK, NSEG), 1)
        oh = (seg == iota).astype(jnp.float32)
        pooled[...] += lax.dot_general(oh, out2, (((0,), (0,)), ((), ())),
                                       precision=_HI)
        cnts[...] += lax.dot_general(oh, jnp.ones((BLK, 1), jnp.float32),
                                     (((0,), (0,)), ((), ())), precision=_HI)

        @pl.when(i == GRID - 1)
        def _():
            mean = pooled[...] / jnp.maximum(cnts[...], 1.0)
            out_ref[...] = jnp.dot(mean, w_ref[...], precision=_HI) \
                + bias_ref[...]

    return pl.pallas_call(
        body,
        grid=(GRID,),
        in_specs=[
            pl.BlockSpec((2, BLK, F1), lambda i: (0, i, 0)),
            pl.BlockSpec((BLK, F2), lambda i: (i, 0)),
            pl.BlockSpec((BLK, 1), lambda i: (i, 0)),
            pl.BlockSpec((BLK, 1), lambda i: (i, 0)),
            pl.BlockSpec((F2, F2), lambda i: (0, 0)),
            pl.BlockSpec((1, F2), lambda i: (0, 0)),
        ],
        out_specs=pl.BlockSpec((NSEG, F2), lambda i: (0, 0)),
        out_shape=jax.ShapeDtypeStruct((NSEG, F2), jnp.float32),
        scratch_shapes=[
            pltpu.VMEM((NSEG, F2), jnp.float32),
            pltpu.VMEM((NSEG, 1), jnp.float32),
        ],
    )(acc2, self2, dinv, batch2d, W3, b3)


def kernel(x, edge_index, batch, W1, b1, W2, b2, W3, b3):
    ei3 = edge_index.reshape(2, NROWS, ROW)
    ones_row = jnp.ones((ROW, F1), jnp.bfloat16)
    zslice1 = jnp.zeros((SLICE, F1), jnp.bfloat16)
    zslice16 = zslice1

    cnt2 = _sc_count(ei3, ones_row, zslice1)
    dinv, g1, self1 = _tc_dense1(cnt2, x, W1, b1.reshape(1, F1))
    acc1 = _sc_aggregate1(g1, ei3, zslice16)
    g2lo, g2hi, self2 = _tc_dense2(acc1, self1, dinv, W2, b2.reshape(1, F2))
    acc2 = _sc_aggregate2(g2lo, g2hi, ei3, zslice16)
    out = _tc_dense3(acc2, self2, dinv, batch.reshape(N, 1), W3,
                     b3.reshape(1, F2))
    return out


# speedup vs baseline: 31.1358x; 1.2362x over previous
import functools

import jax
import jax.numpy as jnp
from jax import lax
from jax.experimental import pallas as pl
from jax.experimental.pallas import tpu as pltpu
from jax.experimental.pallas import tpu_sc as plsc

N = 100000
E = 1600000
NSEG = 256
F1 = 16
F2 = 32

NTEC = 16
NPAD = 100096
SLICE = NPAD // NTEC
ROW = 80
NROWS = E // ROW

BLK = 4000
GRID = N // BLK


def _mesh():
    return plsc.VectorSubcoreMesh(core_axis_name="c", subcore_axis_name="s")


_SC_PARAMS = pltpu.CompilerParams(use_tc_tiling_on_sc=False)


def _sc_count(ei3, ones_row, zslice1):
    @functools.partial(
        pl.kernel,
        out_type=jax.ShapeDtypeStruct((2, NPAD, F1), jnp.bfloat16),
        mesh=_mesh(),
        compiler_params=_SC_PARAMS,
        scratch_types=[
            pltpu.VMEM((25, ROW), jnp.int32),
            pltpu.VMEM((ROW, F1), jnp.bfloat16),
            pltpu.VMEM_SHARED((NPAD, F1), jnp.bfloat16),
            pltpu.SemaphoreType.DMA,
        ],
    )
    def k(ei_hbm, ones_hbm, z_hbm, out_hbm, di_v, ones_v, cnt_sh, sem2):
        c = lax.axis_index("c")
        s = lax.axis_index("s")
        pltpu.sync_copy(ones_hbm, ones_v)
        pltpu.sync_copy(z_hbm, cnt_sh.at[pl.ds(s * SLICE, SLICE), :])
        plsc.subcore_barrier()
        base = (c * NTEC + s) * (NROWS // 32)

        def outer(t, carry):
            pltpu.sync_copy(ei_hbm.at[1, pl.ds(base + t * 25, 25), :], di_v)

            def inner(u, carry2):
                scs = [pltpu.async_copy(ones_v, cnt_sh.at[di_v.at[u * 5 + j]],
                                        sem2, add=True) for j in range(5)]
                for sc in scs:
                    sc.wait()
                return carry2

            lax.fori_loop(0, 5, inner, 0)
            return carry

        lax.fori_loop(0, (NROWS // 32) // 25, outer, 0)
        plsc.subcore_barrier()
        pltpu.sync_copy(cnt_sh.at[pl.ds(s * SLICE, SLICE), :],
                        out_hbm.at[c, pl.ds(s * SLICE, SLICE), :])

    return k(ei3, ones_row, zslice1)


def _sc_aggregate1(g1, ei3, zslice16):
    @functools.partial(
        pl.kernel,
        out_type=jax.ShapeDtypeStruct((2, NPAD, F1), jnp.bfloat16),
        mesh=_mesh(),
        compiler_params=_SC_PARAMS,
        scratch_types=[
            pltpu.VMEM((25, ROW), jnp.int32),
            pltpu.VMEM((25, ROW), jnp.int32),
            pltpu.VMEM((5, ROW, F1), jnp.bfloat16),
            pltpu.VMEM_SHARED((NPAD, F1), jnp.bfloat16),
            pltpu.SemaphoreType.DMA,
            pltpu.SemaphoreType.DMA,
        ],
    )
    def k(g_hbm, ei_hbm, z_hbm, out_hbm,
          si_v, di_v, rows_v, acc_sh, sem, sem2):
        c = lax.axis_index("c")
        s = lax.axis_index("s")
        pltpu.sync_copy(z_hbm, acc_sh.at[pl.ds(s * SLICE, SLICE), :])
        plsc.subcore_barrier()
        base = (c * NTEC + s) * (NROWS // 32)

        def outer(t, carry):
            r0 = base + t * 25
            pltpu.sync_copy(ei_hbm.at[0, pl.ds(r0, 25), :], si_v)
            pltpu.sync_copy(ei_hbm.at[1, pl.ds(r0, 25), :], di_v)

            def inner(u, carry2):
                cps = [pltpu.async_copy(g_hbm.at[si_v.at[u * 5 + j]],
                                        rows_v.at[j], sem) for j in range(5)]
                scs = []
                for j in range(5):
                    cps[j].wait()
                    scs.append(pltpu.async_copy(rows_v.at[j],
                                                acc_sh.at[di_v.at[u * 5 + j]],
                                                sem2, add=True))
                for sc in scs:
                    sc.wait()
                return carry2

            lax.fori_loop(0, 5, inner, 0)
            return carry

        lax.fori_loop(0, (NROWS // 32) // 25, outer, 0)
        plsc.subcore_barrier()
        pltpu.sync_copy(acc_sh.at[pl.ds(s * SLICE, SLICE), :],
                        out_hbm.at[c, pl.ds(s * SLICE, SLICE), :])

    return k(g1, ei3, zslice16)


def _sc_aggregate2(g2lo, g2hi, ei3, zslice16):
    @functools.partial(
        pl.kernel,
        out_type=jax.ShapeDtypeStruct((2, NPAD, F1), jnp.bfloat16),
        mesh=_mesh(),
        compiler_params=_SC_PARAMS,
        scratch_types=[
            pltpu.VMEM((25, ROW), jnp.int32),
            pltpu.VMEM((25, ROW), jnp.int32),
            pltpu.VMEM((5, ROW, F1), jnp.bfloat16),
            pltpu.VMEM_SHARED((NPAD, F1), jnp.bfloat16),
            pltpu.SemaphoreType.DMA,
            pltpu.SemaphoreType.DMA,
        ],
    )
    def k(glo_hbm, ghi_hbm, ei_hbm, z_hbm, out_hbm,
          si_v, di_v, rows_v, acc_sh, sem, sem2):
        c = lax.axis_index("c")
        s = lax.axis_index("s")
        pltpu.sync_copy(z_hbm, acc_sh.at[pl.ds(s * SLICE, SLICE), :])
        plsc.subcore_barrier()
        base = s * (NROWS // NTEC)

        def outer(t, carry):
            r0 = base + t * 25
            pltpu.sync_copy(ei_hbm.at[0, pl.ds(r0, 25), :], si_v)
            pltpu.sync_copy(ei_hbm.at[1, pl.ds(r0, 25), :], di_v)

            def inner(u, carry2):
                @pl.when(c == 0)
                def _():
                    cps = [pltpu.async_copy(glo_hbm.at[si_v.at[u * 5 + j]],
                                            rows_v.at[j], sem)
                           for j in range(5)]
                    scs = []
                    for j in range(5):
                        cps[j].wait()
                        scs.append(pltpu.async_copy(
                            rows_v.at[j], acc_sh.at[di_v.at[u * 5 + j]],
                            sem2, add=True))
                    for sc in scs:
                        sc.wait()

                @pl.when(c == 1)
                def _():
                    cps = [pltpu.async_copy(ghi_hbm.at[si_v.at[u * 5 + j]],
                                            rows_v.at[j], sem)
                           for j in range(5)]
                    scs = []
                    for j in range(5):
                        cps[j].wait()
                        scs.append(pltpu.async_copy(
                            rows_v.at[j], acc_sh.at[di_v.at[u * 5 + j]],
                            sem2, add=True))
                    for sc in scs:
                        sc.wait()

                return carry2

            lax.fori_loop(0, 5, inner, 0)
            return carry

        lax.fori_loop(0, (NROWS // NTEC) // 25, outer, 0)
        plsc.subcore_barrier()
        pltpu.sync_copy(acc_sh.at[pl.ds(s * SLICE, SLICE), :],
                        out_hbm.at[c, pl.ds(s * SLICE, SLICE), :])

    return k(g2lo, g2hi, ei3, zslice16)


_HI = jax.lax.Precision.HIGHEST


def _tc_dense1(cnt2, x, W1, b1):
    def body(cnt_ref, x_ref, w_ref, b_ref, dinv_ref, g_ref, self_ref):
        deg = (cnt_ref[0, :, 0:1].astype(jnp.float32)
               + cnt_ref[1, :, 0:1].astype(jnp.float32) + 2.0)
        dinv = lax.rsqrt(deg)
        h1 = jnp.dot(x_ref[...], w_ref[...], precision=_HI)
        dinv_ref[...] = dinv
        g_ref[...] = (h1 * dinv).astype(jnp.bfloat16)
        self_ref[...] = (2.0 * dinv * dinv) * h1 + b_ref[...]

    return pl.pallas_call(
        body,
        grid=(GRID,),
        in_specs=[
            pl.BlockSpec((2, BLK, F1), lambda i: (0, i, 0)),
            pl.BlockSpec((BLK, 3), lambda i: (i, 0)),
            pl.BlockSpec((3, F1), lambda i: (0, 0)),
            pl.BlockSpec((1, F1), lambda i: (0, 0)),
        ],
        out_specs=[
            pl.BlockSpec((BLK, 1), lambda i: (i, 0)),
            pl.BlockSpec((BLK, F1), lambda i: (i, 0)),
            pl.BlockSpec((BLK, F1), lambda i: (i, 0)),
        ],
        out_shape=[
            jax.ShapeDtypeStruct((N, 1), jnp.float32),
            jax.ShapeDtypeStruct((N, F1), jnp.bfloat16),
            jax.ShapeDtypeStruct((N, F1), jnp.float32),
        ],
    )(cnt2, x, W1, b1)


def _tc_dense2(acc1, self1, dinv, W2, b2):
    def body(a_ref, s_ref, d_ref, w_ref, b_ref, glo_ref, ghi_ref, self_ref):
        a = a_ref[0].astype(jnp.float32) \
            + a_ref[1].astype(jnp.float32)
        dinv = d_ref[...]
        x2 = jnp.maximum(dinv * a + s_ref[...], 0.0)
        h2 = jnp.dot(x2, w_ref[...], precision=_HI)
        g2 = (h2 * dinv).astype(jnp.bfloat16)
        glo_ref[...] = g2[:, :F1]
        ghi_ref[...] = g2[:, F1:]
        self_ref[...] = (2.0 * dinv * dinv) * h2 + b_ref[...]

    return pl.pallas_call(
        body,
        grid=(GRID,),
        in_specs=[
            pl.BlockSpec((2, BLK, F1), lambda i: (0, i, 0)),
            pl.BlockSpec((BLK, F1), lambda i: (i, 0)),
            pl.BlockSpec((BLK, 1), lambda i: (i, 0)),
            pl.BlockSpec((F1, F2), lambda i: (0, 0)),
            pl.BlockSpec((1, F2), lambda i: (0, 0)),
        ],
        out_specs=[
            pl.BlockSpec((BLK, F1), lambda i: (i, 0)),
            pl.BlockSpec((BLK, F1), lambda i: (i, 0)),
            pl.BlockSpec((BLK, F2), lambda i: (i, 0)),
        ],
        out_shape=[
            jax.ShapeDtypeStruct((N, F1), jnp.bfloat16),
            jax.ShapeDtypeStruct((N, F1), jnp.bfloat16),
            jax.ShapeDtypeStruct((N, F2), jnp.float32),
        ],
    )(acc1, self1, dinv, W2, b2)


def _tc_dense3(acc2, self2, dinv, batch2d, W3, b3):
    def body(a_ref, s_ref, d_ref, b_ref, w_ref, bias_ref, out_ref,
             pooled, cnts):
        i = pl.program_id(0)

        @pl.when(i == 0)
        def _():
            pooled[...] = jnp.zeros_like(pooled)
            cnts[...] = jnp.zeros_like(cnts)

        out2 = jnp.maximum(
            d_ref[...] * jnp.concatenate(
                [a_ref[0].astype(jnp.float32), a_ref[1].astype(jnp.float32)],
                axis=1)
            + s_ref[...], 0.0)
        seg = b_ref[...]
        iota = lax.broadcasted_iota(jnp.int32, (BLK, NSEG), 1)
        oh = (seg == iota).astype(jnp.float32)
        pooled[...] += lax.dot_general(oh, out2, (((0,), (0,)), ((), ())),
                                       precision=_HI)
        cnts[...] += lax.dot_general(oh, jnp.ones((BLK, 1), jnp.float32),
                                     (((0,), (0,)), ((), ())), precision=_HI)

        @pl.when(i == GRID - 1)
        def _():
            mean = pooled[...] / jnp.maximum(cnts[...], 1.0)
            out_ref[...] = jnp.dot(mean, w_ref[...], precision=_HI) \
                + bias_ref[...]

    return pl.pallas_call(
        body,
        grid=(GRID,),
        in_specs=[
            pl.BlockSpec((2, BLK, F1), lambda i: (0, i, 0)),
            pl.BlockSpec((BLK, F2), lambda i: (i, 0)),
            pl.BlockSpec((BLK, 1), lambda i: (i, 0)),
            pl.BlockSpec((BLK, 1), lambda i: (i, 0)),
            pl.BlockSpec((F2, F2), lambda i: (0, 0)),
            pl.BlockSpec((1, F2), lambda i: (0, 0)),
        ],
        out_specs=pl.BlockSpec((NSEG, F2), lambda i: (0, 0)),
        out_shape=jax.ShapeDtypeStruct((NSEG, F2), jnp.float32),
        scratch_shapes=[
            pltpu.VMEM((NSEG, F2), jnp.float32),
            pltpu.VMEM((NSEG, 1), jnp.float32),
        ],
    )(acc2, self2, dinv, batch2d, W3, b3)


def kernel(x, edge_index, batch, W1, b1, W2, b2, W3, b3):
    ei3 = edge_index.reshape(2, NROWS, ROW)
    ones_row = jnp.ones((ROW, F1), jnp.bfloat16)
    zslice1 = jnp.zeros((SLICE, F1), jnp.bfloat16)
    zslice16 = zslice1

    cnt2 = _sc_count(ei3, ones_row, zslice1)
    dinv, g1, self1 = _tc_dense1(cnt2, x, W1, b1.reshape(1, F1))
    acc1 = _sc_aggregate1(g1, ei3, zslice16)
    g2lo, g2hi, self2 = _tc_dense2(acc1, self1, dinv, W2, b2.reshape(1, F2))
    acc2 = _sc_aggregate2(g2lo, g2hi, ei3, zslice16)
    out = _tc_dense3(acc2, self2, dinv, batch.reshape(N, 1), W3,
                     b3.reshape(1, F2))
    return out


# speedup vs baseline: 32.9551x; 1.0584x over previous
import functools

import jax
import jax.numpy as jnp
from jax import lax
from jax.experimental import pallas as pl
from jax.experimental.pallas import tpu as pltpu
from jax.experimental.pallas import tpu_sc as plsc

N = 100000
E = 1600000
NSEG = 256
F1 = 16
F2 = 32

NTEC = 16
NPAD = 100096
SLICE = NPAD // NTEC
ROW = 80
NROWS = E // ROW

BLK = 4000
GRID = N // BLK


def _mesh():
    return plsc.VectorSubcoreMesh(core_axis_name="c", subcore_axis_name="s")


_SC_PARAMS = pltpu.CompilerParams(use_tc_tiling_on_sc=False)


def _sc_count(ei3, ones_row, zslice1):
    @functools.partial(
        pl.kernel,
        out_type=jax.ShapeDtypeStruct((2, NPAD, F1), jnp.bfloat16),
        mesh=_mesh(),
        compiler_params=_SC_PARAMS,
        scratch_types=[
            pltpu.VMEM((125, ROW), jnp.int32),
            pltpu.VMEM((ROW, F1), jnp.bfloat16),
            pltpu.VMEM_SHARED((NPAD, F1), jnp.bfloat16),
            pltpu.SemaphoreType.DMA,
        ],
    )
    def k(ei_hbm, ones_hbm, z_hbm, out_hbm, di_v, ones_v, cnt_sh, sem2):
        c = lax.axis_index("c")
        s = lax.axis_index("s")
        pltpu.sync_copy(ones_hbm, ones_v)
        pltpu.sync_copy(z_hbm, cnt_sh.at[pl.ds(s * SLICE, SLICE), :])
        plsc.subcore_barrier()
        base = (c * NTEC + s) * (NROWS // 32)

        def outer(t, carry):
            pltpu.sync_copy(ei_hbm.at[1, pl.ds(base + t * 125, 125), :], di_v)

            def inner(u, carry2):
                scs = [pltpu.async_copy(ones_v, cnt_sh.at[di_v.at[u * 5 + j]],
                                        sem2, add=True) for j in range(5)]
                for sc in scs:
                    sc.wait()
                return carry2

            lax.fori_loop(0, 25, inner, 0)
            return carry

        lax.fori_loop(0, (NROWS // 32) // 125, outer, 0)
        plsc.subcore_barrier()
        pltpu.sync_copy(cnt_sh.at[pl.ds(s * SLICE, SLICE), :],
                        out_hbm.at[c, pl.ds(s * SLICE, SLICE), :])

    return k(ei3, ones_row, zslice1)


def _sc_aggregate1(g1, ei3, zslice16):
    @functools.partial(
        pl.kernel,
        out_type=jax.ShapeDtypeStruct((2, NPAD, F1), jnp.bfloat16),
        mesh=_mesh(),
        compiler_params=_SC_PARAMS,
        scratch_types=[
            pltpu.VMEM((125, ROW), jnp.int32),
            pltpu.VMEM((125, ROW), jnp.int32),
            pltpu.VMEM((5, ROW, F1), jnp.bfloat16),
            pltpu.VMEM_SHARED((NPAD, F1), jnp.bfloat16),
            pltpu.SemaphoreType.DMA,
            pltpu.SemaphoreType.DMA,
        ],
    )
    def k(g_hbm, ei_hbm, z_hbm, out_hbm,
          si_v, di_v, rows_v, acc_sh, sem, sem2):
        c = lax.axis_index("c")
        s = lax.axis_index("s")
        pltpu.sync_copy(z_hbm, acc_sh.at[pl.ds(s * SLICE, SLICE), :])
        plsc.subcore_barrier()
        base = (c * NTEC + s) * (NROWS // 32)

        def outer(t, carry):
            r0 = base + t * 125
            pltpu.sync_copy(ei_hbm.at[0, pl.ds(r0, 125), :], si_v)
            pltpu.sync_copy(ei_hbm.at[1, pl.ds(r0, 125), :], di_v)

            def inner(u, carry2):
                cps = [pltpu.async_copy(g_hbm.at[si_v.at[u * 5 + j]],
                                        rows_v.at[j], sem) for j in range(5)]
                scs = []
                for j in range(5):
                    cps[j].wait()
                    scs.append(pltpu.async_copy(rows_v.at[j],
                                                acc_sh.at[di_v.at[u * 5 + j]],
                                                sem2, add=True))
                for sc in scs:
                    sc.wait()
                return carry2

            lax.fori_loop(0, 25, inner, 0)
            return carry

        lax.fori_loop(0, (NROWS // 32) // 125, outer, 0)
        plsc.subcore_barrier()
        pltpu.sync_copy(acc_sh.at[pl.ds(s * SLICE, SLICE), :],
                        out_hbm.at[c, pl.ds(s * SLICE, SLICE), :])

    return k(g1, ei3, zslice16)


def _sc_aggregate2(g2lo, g2hi, ei3, zslice16):
    @functools.partial(
        pl.kernel,
        out_type=jax.ShapeDtypeStruct((2, NPAD, F1), jnp.bfloat16),
        mesh=_mesh(),
        compiler_params=_SC_PARAMS,
        scratch_types=[
            pltpu.VMEM((125, ROW), jnp.int32),
            pltpu.VMEM((125, ROW), jnp.int32),
            pltpu.VMEM((5, ROW, F1), jnp.bfloat16),
            pltpu.VMEM_SHARED((NPAD, F1), jnp.bfloat16),
            pltpu.SemaphoreType.DMA,
            pltpu.SemaphoreType.DMA,
        ],
    )
    def k(glo_hbm, ghi_hbm, ei_hbm, z_hbm, out_hbm,
          si_v, di_v, rows_v, acc_sh, sem, sem2):
        c = lax.axis_index("c")
        s = lax.axis_index("s")
        pltpu.sync_copy(z_hbm, acc_sh.at[pl.ds(s * SLICE, SLICE), :])
        plsc.subcore_barrier()
        base = s * (NROWS // NTEC)

        def outer(t, carry):
            r0 = base + t * 125
            pltpu.sync_copy(ei_hbm.at[0, pl.ds(r0, 125), :], si_v)
            pltpu.sync_copy(ei_hbm.at[1, pl.ds(r0, 125), :], di_v)

            def inner(u, carry2):
                @pl.when(c == 0)
                def _():
                    cps = [pltpu.async_copy(glo_hbm.at[si_v.at[u * 5 + j]],
                                            rows_v.at[j], sem)
                           for j in range(5)]
                    scs = []
                    for j in range(5):
                        cps[j].wait()
                        scs.append(pltpu.async_copy(
                            rows_v.at[j], acc_sh.at[di_v.at[u * 5 + j]],
                            sem2, add=True))
                    for sc in scs:
                        sc.wait()

                @pl.when(c == 1)
                def _():
                    cps = [pltpu.async_copy(ghi_hbm.at[si_v.at[u * 5 + j]],
                                            rows_v.at[j], sem)
                           for j in range(5)]
                    scs = []
                    for j in range(5):
                        cps[j].wait()
                        scs.append(pltpu.async_copy(
                            rows_v.at[j], acc_sh.at[di_v.at[u * 5 + j]],
                            sem2, add=True))
                    for sc in scs:
                        sc.wait()

                return carry2

            lax.fori_loop(0, 25, inner, 0)
            return carry

        lax.fori_loop(0, (NROWS // NTEC) // 125, outer, 0)
        plsc.subcore_barrier()
        pltpu.sync_copy(acc_sh.at[pl.ds(s * SLICE, SLICE), :],
                        out_hbm.at[c, pl.ds(s * SLICE, SLICE), :])

    return k(g2lo, g2hi, ei3, zslice16)


_HI = jax.lax.Precision.HIGHEST


def _tc_dense1(cnt2, x, W1, b1):
    def body(cnt_ref, x_ref, w_ref, b_ref, dinv_ref, g_ref, self_ref):
        deg = (cnt_ref[0, :, 0:1].astype(jnp.float32)
               + cnt_ref[1, :, 0:1].astype(jnp.float32) + 2.0)
        dinv = lax.rsqrt(deg)
        h1 = jnp.dot(x_ref[...], w_ref[...], precision=_HI)
        dinv_ref[...] = dinv
        g_ref[...] = (h1 * dinv).astype(jnp.bfloat16)
        self_ref[...] = (2.0 * dinv * dinv) * h1 + b_ref[...]

    return pl.pallas_call(
        body,
        grid=(GRID,),
        in_specs=[
            pl.BlockSpec((2, BLK, F1), lambda i: (0, i, 0)),
            pl.BlockSpec((BLK, 3), lambda i: (i, 0)),
            pl.BlockSpec((3, F1), lambda i: (0, 0)),
            pl.BlockSpec((1, F1), lambda i: (0, 0)),
        ],
        out_specs=[
            pl.BlockSpec((BLK, 1), lambda i: (i, 0)),
            pl.BlockSpec((BLK, F1), lambda i: (i, 0)),
            pl.BlockSpec((BLK, F1), lambda i: (i, 0)),
        ],
        out_shape=[
            jax.ShapeDtypeStruct((N, 1), jnp.float32),
            jax.ShapeDtypeStruct((N, F1), jnp.bfloat16),
            jax.ShapeDtypeStruct((N, F1), jnp.float32),
        ],
    )(cnt2, x, W1, b1)


def _tc_dense2(acc1, self1, dinv, W2, b2):
    def body(a_ref, s_ref, d_ref, w_ref, b_ref, glo_ref, ghi_ref, self_ref):
        a = a_ref[0].astype(jnp.float32) \
            + a_ref[1].astype(jnp.float32)
        dinv = d_ref[...]
        x2 = jnp.maximum(dinv * a + s_ref[...], 0.0)
        h2 = jnp.dot(x2, w_ref[...], precision=_HI)
        g2 = (h2 * dinv).astype(jnp.bfloat16)
        glo_ref[...] = g2[:, :F1]
        ghi_ref[...] = g2[:, F1:]
        self_ref[...] = (2.0 * dinv * dinv) * h2 + b_ref[...]

    return pl.pallas_call(
        body,
        grid=(GRID,),
        in_specs=[
            pl.BlockSpec((2, BLK, F1), lambda i: (0, i, 0)),
            pl.BlockSpec((BLK, F1), lambda i: (i, 0)),
            pl.BlockSpec((BLK, 1), lambda i: (i, 0)),
            pl.BlockSpec((F1, F2), lambda i: (0, 0)),
            pl.BlockSpec((1, F2), lambda i: (0, 0)),
        ],
        out_specs=[
            pl.BlockSpec((BLK, F1), lambda i: (i, 0)),
            pl.BlockSpec((BLK, F1), lambda i: (i, 0)),
            pl.BlockSpec((BLK, F2), lambda i: (i, 0)),
        ],
        out_shape=[
            jax.ShapeDtypeStruct((N, F1), jnp.bfloat16),
            jax.ShapeDtypeStruct((N, F1), jnp.bfloat16),
            jax.ShapeDtypeStruct((N, F2), jnp.float32),
        ],
    )(acc1, self1, dinv, W2, b2)


def _tc_dense3(acc2, self2, dinv, batch2d, W3, b3):
    def body(a_ref, s_ref, d_ref, b_ref, w_ref, bias_ref, out_ref,
             pooled, cnts):
        i = pl.program_id(0)

        @pl.when(i == 0)
        def _():
            pooled[...] = jnp.zeros_like(pooled)
            cnts[...] = jnp.zeros_like(cnts)

        out2 = jnp.maximum(
            d_ref[...] * jnp.concatenate(
                [a_ref[0].astype(jnp.float32), a_ref[1].astype(jnp.float32)],
                axis=1)
            + s_ref[...], 0.0)
        seg = b_ref[...]
        iota = lax.broadcasted_iota(jnp.int32, (BLK, NSEG), 1)
        oh = (seg == iota).astype(jnp.float32)
        pooled[...] += lax.dot_general(oh, out2, (((0,), (0,)), ((), ())),
                                       precision=_HI)
        cnts[...] += lax.dot_general(oh, jnp.ones((BLK, 1), jnp.float32),
                                     (((0,), (0,)), ((), ())), precision=_HI)

        @pl.when(i == GRID - 1)
        def _():
            mean = pooled[...] / jnp.maximum(cnts[...], 1.0)
            out_ref[...] = jnp.dot(mean, w_ref[...], precision=_HI) \
                + bias_ref[...]

    return pl.pallas_call(
        body,
        grid=(GRID,),
        in_specs=[
            pl.BlockSpec((2, BLK, F1), lambda i: (0, i, 0)),
            pl.BlockSpec((BLK, F2), lambda i: (i, 0)),
            pl.BlockSpec((BLK, 1), lambda i: (i, 0)),
            pl.BlockSpec((BLK, 1), lambda i: (i, 0)),
            pl.BlockSpec((F2, F2), lambda i: (0, 0)),
            pl.BlockSpec((1, F2), lambda i: (0, 0)),
        ],
        out_specs=pl.BlockSpec((NSEG, F2), lambda i: (0, 0)),
        out_shape=jax.ShapeDtypeStruct((NSEG, F2), jnp.float32),
        scratch_shapes=[
            pltpu.VMEM((NSEG, F2), jnp.float32),
            pltpu.VMEM((NSEG, 1), jnp.float32),
        ],
    )(acc2, self2, dinv, batch2d, W3, b3)


def kernel(x, edge_index, batch, W1, b1, W2, b2, W3, b3):
    ei3 = edge_index.reshape(2, NROWS, ROW)
    ones_row = jnp.ones((ROW, F1), jnp.bfloat16)
    zslice1 = jnp.zeros((SLICE, F1), jnp.bfloat16)
    zslice16 = zslice1

    cnt2 = _sc_count(ei3, ones_row, zslice1)
    dinv, g1, self1 = _tc_dense1(cnt2, x, W1, b1.reshape(1, F1))
    acc1 = _sc_aggregate1(g1, ei3, zslice16)
    g2lo, g2hi, self2 = _tc_dense2(acc1, self1, dinv, W2, b2.reshape(1, F2))
    acc2 = _sc_aggregate2(g2lo, g2hi, ei3, zslice16)
    out = _tc_dense3(acc2, self2, dinv, batch.reshape(N, 1), W3,
                     b3.reshape(1, F2))
    return out
